# baseline (speedup 1.0000x reference)
import functools

import jax
import jax.numpy as jnp
from jax import lax
from jax.experimental import pallas as pl
from jax.experimental.pallas import tpu as pltpu
from jax.experimental.pallas import tpu_sc as plsc

_NC = 2
_NS = 16
_NW = _NC * _NS


@jax.jit
def _gather(labels, tbl3):
    batch = labels.shape[0]
    b_per_w = batch // _NW
    dim = tbl3.shape[2]
    mesh = plsc.VectorSubcoreMesh(core_axis_name="c", subcore_axis_name="s")

    @functools.partial(
        pl.kernel,
        out_type=jax.ShapeDtypeStruct((batch // 8, 8, dim), jnp.float32),
        mesh=mesh,
        scratch_types=[
            pltpu.VMEM((b_per_w,), jnp.int32),
            pltpu.SemaphoreType.DMA,
        ],
        compiler_params=pltpu.CompilerParams(needs_layout_passes=False),
    )
    def k(tbl_hbm, lab_hbm, out_hbm, lab_v, sem):
        wid = lax.axis_index("s") * _NC + lax.axis_index("c")
        base = wid * b_per_w
        pltpu.sync_copy(lab_hbm.at[pl.ds(base, b_per_w)], lab_v)

        copies = []
        for g in range(b_per_w // 16):
            labv = lab_v[pl.ds(g * 16, 16)]
            for i in range(16):
                lab = labv[i]
                blk = lax.shift_right_logical(lab, 3)
                sel = lax.bitwise_and(lab, 7)
                idx = base + g * 16 + i
                copies.append(
                    pltpu.async_copy(
                        tbl_hbm.at[blk, sel],
                        out_hbm.at[idx // 8, idx % 8],
                        sem,
                    )
                )
        for c in copies:
            c.wait()

    return k(tbl3, labels)


def kernel(batch_size, class_labels, class_embedding):
    labels = class_labels.astype(jnp.int32)
    tbl3 = class_embedding.reshape(-1, 8, class_embedding.shape[1])
    out = _gather(labels, tbl3)
    return out.reshape(-1, class_embedding.shape[1])
